# baseline (speedup 1.0000x reference)
import jax
import jax.numpy as jnp
from jax import lax
from jax.experimental import pallas as pl
from jax.experimental.pallas import tpu as pltpu, tpu_sc as plsc

VOCAB = 1000000
EMBED_DIM = 64
BATCH = 16384
FIELDS = 26

NC = 2
NS = 16
NW = NC * NS

CHUNK = 128
NCHUNK = BATCH // CHUNK
TASKS = FIELDS * NCHUNK
TASKS_PER_W = TASKS // NW
STEPS = TASKS_PER_W // 2


def _embed_kernel(idx_hbm, table_hbm, out_hbm, ib, g0, g1, t0b, t1b, gsems, wsems):
    gs = [g0, g1]
    ts = [t0b, t1b]
    wid = lax.axis_index("s") * NC + lax.axis_index("c")
    t0 = wid * TASKS_PER_W

    def task_fc(k):
        t = t0 + k
        return t // NCHUNK, t % NCHUNK

    def stage_and_fire(k, b):
        f, c = task_fc(k)
        pltpu.sync_copy(idx_hbm.at[f, pl.ds(c * CHUNK, CHUNK)], ib.at[b])
        pltpu.async_copy(table_hbm.at[ib.at[b]], gs[b], gsems.at[b])

    for b in range(2):
        stage_and_fire(b, b)

    iota = lax.broadcasted_iota(jnp.int32, (16,), 0)
    a_idx = [(d0 + iota) >> 3 for d0 in range(0, EMBED_DIM, 16)]
    s_idx = [(d0 + iota) & 7 for d0 in range(0, EMBED_DIM, 16)]

    def transpose_block(b):
        for l in range(CHUNK):
            col = jnp.full((16,), l, jnp.int32)
            for q in range(EMBED_DIM // 16):
                x = gs[b][l, pl.ds(q * 16, 16)]
                plsc.store_scatter(ts[b], [a_idx[q], s_idx[q], col], x)

    def wb_dst(f, c):
        return out_hbm.at[f, :, c]

    def step(i, carry):
        for b in range(2):
            k = i * 2 + b
            f, c = task_fc(k)
            @pl.when(i > 0)
            def _():
                pltpu.make_async_copy(
                    ts[b].at[:, :, pl.ds(0, CHUNK)], wb_dst(f, c),
                    wsems.at[b]).wait()
            pltpu.make_async_copy(
                table_hbm.at[ib.at[b]], gs[b], gsems.at[b]).wait()
            transpose_block(b)
            pltpu.async_copy(ts[b].at[:, :, pl.ds(0, CHUNK)], wb_dst(f, c),
                             wsems.at[b])
            @pl.when(i < STEPS - 1)
            def _():
                stage_and_fire(k + 2, b)
        return carry

    lax.fori_loop(0, STEPS, step, 0)
    for b in range(2):
        k = (STEPS - 1) * 2 + b
        f, c = task_fc(k)
        pltpu.make_async_copy(
            ts[b].at[:, :, pl.ds(0, CHUNK)], wb_dst(f, c), wsems.at[b]).wait()


PAD_DIM = 96


def kernel(embed_input, weight):
    idx_t = embed_input.T
    w_pad = jnp.pad(weight, ((0, 0), (0, PAD_DIM - EMBED_DIM)))
    mesh = plsc.VectorSubcoreMesh(core_axis_name="c", subcore_axis_name="s")
    o5 = pl.kernel(
        _embed_kernel,
        out_type=jax.ShapeDtypeStruct((FIELDS, 8, NCHUNK, 8, CHUNK),
                                      jnp.float32),
        mesh=mesh,
        compiler_params=pltpu.CompilerParams(use_tc_tiling_on_sc=False,
                                             needs_layout_passes=False),
        scratch_types=[
            pltpu.VMEM((2, CHUNK), jnp.int32),
            pltpu.VMEM((CHUNK, 96), jnp.float32),
            pltpu.VMEM((CHUNK, 96), jnp.float32),
            pltpu.VMEM((8, 8, CHUNK + 1), jnp.float32),
            pltpu.VMEM((8, 8, CHUNK + 1), jnp.float32),
            pltpu.SemaphoreType.DMA((2,)),
            pltpu.SemaphoreType.DMA((2,)),
        ],
    )(idx_t, w_pad)
    return o5.transpose(2, 4, 0, 1, 3).reshape(BATCH, FIELDS, EMBED_DIM)


# speedup vs baseline: 1.6032x; 1.6032x over previous
import jax
import jax.numpy as jnp
from jax import lax
from jax.experimental import pallas as pl
from jax.experimental.pallas import tpu as pltpu, tpu_sc as plsc

VOCAB = 1000000
EMBED_DIM = 64
BATCH = 16384
FIELDS = 26

NC = 2
NS = 16
NW = NC * NS

CHUNK = 128
NCHUNK = BATCH // CHUNK
TASKS = FIELDS * NCHUNK
TASKS_PER_W = TASKS // NW
STEPS = TASKS_PER_W // 2


def _embed_kernel(idx_hbm, table_hbm, out_hbm, ib, g0, g1, t0b, t1b, gsems, wsems):
    gs = [g0, g1]
    ts = [t0b, t1b]
    wid = lax.axis_index("s") * NC + lax.axis_index("c")
    t0 = wid * TASKS_PER_W

    def task_fc(k):
        t = t0 + k
        return t // NCHUNK, t % NCHUNK

    def stage_and_fire(k, b):
        f, c = task_fc(k)
        pltpu.sync_copy(idx_hbm.at[f, pl.ds(c * CHUNK, CHUNK)], ib.at[b])
        pltpu.async_copy(table_hbm.at[ib.at[b]], gs[b], gsems.at[b])

    for b in range(2):
        stage_and_fire(b, b)

    iota = lax.broadcasted_iota(jnp.int32, (16,), 0)
    a_idx = [(d0 + iota) >> 3 for d0 in range(0, EMBED_DIM, 16)]
    s_idx = [(d0 + iota) & 7 for d0 in range(0, EMBED_DIM, 16)]

    def transpose_block(b):
        for l in range(CHUNK):
            col = jnp.full((16,), l, jnp.int32)
            for q in range(EMBED_DIM // 16):
                x = gs[b][l, pl.ds(q * 16, 16)]
                plsc.store_scatter(ts[b], [a_idx[q], s_idx[q], col], x)

    def wb_dst(f, c):
        return out_hbm.at[f, :, c]

    def step(i, carry):
        for b in range(2):
            k = i * 2 + b
            f, c = task_fc(k)
            @pl.when(i > 0)
            def _():
                pltpu.make_async_copy(
                    ts[b].at[:, :, pl.ds(0, CHUNK)], wb_dst(f, c),
                    wsems.at[b]).wait()
            pltpu.make_async_copy(
                table_hbm.at[ib.at[b]], gs[b], gsems.at[b]).wait()
            transpose_block(b)
            pltpu.async_copy(ts[b].at[:, :, pl.ds(0, CHUNK)], wb_dst(f, c),
                             wsems.at[b])
            @pl.when(i < STEPS - 1)
            def _():
                stage_and_fire(k + 2, b)
        return carry

    lax.fori_loop(0, STEPS, step, 0)
    for b in range(2):
        k = (STEPS - 1) * 2 + b
        f, c = task_fc(k)
        pltpu.make_async_copy(
            ts[b].at[:, :, pl.ds(0, CHUNK)], wb_dst(f, c), wsems.at[b]).wait()


PAD_DIM = 128


def kernel(embed_input, weight):
    idx_t = embed_input.T
    w_pad = jnp.pad(weight, ((0, 0), (0, PAD_DIM - EMBED_DIM)))
    mesh = plsc.VectorSubcoreMesh(core_axis_name="c", subcore_axis_name="s")
    o5 = pl.kernel(
        _embed_kernel,
        out_type=jax.ShapeDtypeStruct((FIELDS, 8, NCHUNK, 8, CHUNK),
                                      jnp.float32),
        mesh=mesh,
        compiler_params=pltpu.CompilerParams(use_tc_tiling_on_sc=False,
                                             needs_layout_passes=False),
        scratch_types=[
            pltpu.VMEM((2, CHUNK), jnp.int32),
            pltpu.VMEM((CHUNK, 128), jnp.float32),
            pltpu.VMEM((CHUNK, 128), jnp.float32),
            pltpu.VMEM((8, 8, CHUNK + 1), jnp.float32),
            pltpu.VMEM((8, 8, CHUNK + 1), jnp.float32),
            pltpu.SemaphoreType.DMA((2,)),
            pltpu.SemaphoreType.DMA((2,)),
        ],
    )(idx_t, w_pad)
    return o5.transpose(2, 4, 0, 1, 3).reshape(BATCH, FIELDS, EMBED_DIM)


# speedup vs baseline: 1.6064x; 1.0020x over previous
import jax
import jax.numpy as jnp
from jax import lax
from jax.experimental import pallas as pl
from jax.experimental.pallas import tpu as pltpu, tpu_sc as plsc

VOCAB = 1000000
EMBED_DIM = 64
BATCH = 16384
FIELDS = 26

NC = 2
NS = 16
NW = NC * NS

CHUNK = 128
NCHUNK = BATCH // CHUNK
TASKS = FIELDS * NCHUNK
TASKS_PER_W = TASKS // NW
STEPS = TASKS_PER_W // 2


def _embed_kernel(idx_hbm, table_hbm, out_hbm, ib, g0, g1, t0b, t1b, gsems, wsems):
    gs = [g0, g1]
    ts = [t0b, t1b]
    wid = lax.axis_index("s") * NC + lax.axis_index("c")
    t0 = wid * TASKS_PER_W

    def task_fc(k):
        t = t0 + k
        return t // NCHUNK, t % NCHUNK

    def stage_and_fire(k, b):
        f, c = task_fc(k)
        pltpu.sync_copy(idx_hbm.at[f, pl.ds(c * CHUNK, CHUNK)], ib.at[b])
        for j in range(CHUNK // 16):
            ib[b, pl.ds(j * 16, 16)] = ib[b, pl.ds(j * 16, 16)] << 1
        pltpu.async_copy(table_hbm.at[ib.at[b]], gs[b], gsems.at[b])

    for b in range(2):
        stage_and_fire(b, b)

    iota = lax.broadcasted_iota(jnp.int32, (16,), 0)
    a_idx = [(d0 + iota) >> 3 for d0 in range(0, EMBED_DIM, 16)]
    s_idx = [(d0 + iota) & 7 for d0 in range(0, EMBED_DIM, 16)]

    def transpose_block(b):
        for l in range(CHUNK):
            col = jnp.full((16,), l, jnp.int32)
            for q in range(EMBED_DIM // 16):
                x = gs[b][l, pl.ds(q * 16, 16)]
                plsc.store_scatter(ts[b], [a_idx[q], s_idx[q], col], x)

    def wb_dst(f, c):
        return out_hbm.at[f, :, c]

    def step(i, carry):
        for b in range(2):
            k = i * 2 + b
            f, c = task_fc(k)
            @pl.when(i > 0)
            def _():
                pltpu.make_async_copy(
                    ts[b].at[:, :, pl.ds(0, CHUNK)], wb_dst(f, c),
                    wsems.at[b]).wait()
            pltpu.make_async_copy(
                table_hbm.at[ib.at[b]], gs[b], gsems.at[b]).wait()
            transpose_block(b)
            pltpu.async_copy(ts[b].at[:, :, pl.ds(0, CHUNK)], wb_dst(f, c),
                             wsems.at[b])
            @pl.when(i < STEPS - 1)
            def _():
                stage_and_fire(k + 2, b)
        return carry

    lax.fori_loop(0, STEPS, step, 0)
    for b in range(2):
        k = (STEPS - 1) * 2 + b
        f, c = task_fc(k)
        pltpu.make_async_copy(
            ts[b].at[:, :, pl.ds(0, CHUNK)], wb_dst(f, c), wsems.at[b]).wait()


PAD_DIM = 128


def kernel(embed_input, weight):
    idx_t = embed_input.T
    w_pad = jnp.pad(weight, ((0, 0), (0, PAD_DIM - EMBED_DIM)))
    w2 = w_pad.reshape(2 * VOCAB, EMBED_DIM)
    mesh = plsc.VectorSubcoreMesh(core_axis_name="c", subcore_axis_name="s")
    o5 = pl.kernel(
        _embed_kernel,
        out_type=jax.ShapeDtypeStruct((FIELDS, 8, NCHUNK, 8, CHUNK),
                                      jnp.float32),
        mesh=mesh,
        compiler_params=pltpu.CompilerParams(use_tc_tiling_on_sc=False,
                                             needs_layout_passes=False),
        scratch_types=[
            pltpu.VMEM((2, CHUNK), jnp.int32),
            pltpu.VMEM((CHUNK, EMBED_DIM), jnp.float32),
            pltpu.VMEM((CHUNK, EMBED_DIM), jnp.float32),
            pltpu.VMEM((8, 8, CHUNK + 1), jnp.float32),
            pltpu.VMEM((8, 8, CHUNK + 1), jnp.float32),
            pltpu.SemaphoreType.DMA((2,)),
            pltpu.SemaphoreType.DMA((2,)),
        ],
    )(idx_t, w2)
    return o5.transpose(2, 4, 0, 1, 3).reshape(BATCH, FIELDS, EMBED_DIM)
